# baseline (speedup 1.0000x reference)
import functools

import jax
import jax.numpy as jnp
from jax import lax
from jax.experimental import pallas as pl
from jax.experimental.pallas import tpu as pltpu
from jax.experimental.pallas import tpu_sc as plsc

_B, _N, _D = 4096, 50, 256
_HID = 2 * _D
_OUT = 256

_NC, _NS = 2, 16
_NW = _NC * _NS
_B_PER_W = _B // _NW


@functools.cache
def _make_sc_gather():
    mesh = plsc.VectorSubcoreMesh(core_axis_name="c", subcore_axis_name="s")

    @functools.partial(
        pl.kernel,
        mesh=mesh,
        out_type=jax.ShapeDtypeStruct((_B, _D), jnp.float32),
        scratch_types=[
            pltpu.VMEM((_B_PER_W,), jnp.int32),
            pltpu.VMEM((_B_PER_W, _D), jnp.float32),
            pltpu.SemaphoreType.DMA,
        ],
    )
    def gather_k(table_hbm, idx_hbm, out_hbm, idx_v, rows_v, sem):
        wid = lax.axis_index("s") * _NC + lax.axis_index("c")
        base = wid * _B_PER_W
        pltpu.sync_copy(idx_hbm.at[pl.ds(base, _B_PER_W)], idx_v)
        pltpu.async_copy(table_hbm.at[idx_v], rows_v, sem).wait()
        pltpu.sync_copy(rows_v, out_hbm.at[pl.ds(base, _B_PER_W)])

    return gather_k


_BB = 256


def _tc_body(x_ref, local_ref, w1_ref, b1_ref, w2_ref, b2_ref, o_ref):
    mean = jnp.mean(x_ref[...], axis=1)
    h = jnp.concatenate([local_ref[...], mean], axis=1)
    h = jnp.dot(h, w1_ref[...], preferred_element_type=jnp.float32)
    h = jnp.maximum(h + b1_ref[...], 0.0)
    h = jnp.dot(h, w2_ref[...], preferred_element_type=jnp.float32)
    o_ref[...] = jnp.maximum(h + b2_ref[...], 0.0)


def _tc_call(x, local, W1, b1, W2, b2):
    grid = (_B // _BB,)
    return pl.pallas_call(
        _tc_body,
        grid=grid,
        in_specs=[
            pl.BlockSpec((_BB, _N, _D), lambda i: (i, 0, 0)),
            pl.BlockSpec((_BB, _D), lambda i: (i, 0)),
            pl.BlockSpec((_HID, _HID), lambda i: (0, 0)),
            pl.BlockSpec((1, _HID), lambda i: (0, 0)),
            pl.BlockSpec((_HID, _OUT), lambda i: (0, 0)),
            pl.BlockSpec((1, _OUT), lambda i: (0, 0)),
        ],
        out_specs=pl.BlockSpec((_BB, _OUT), lambda i: (i, 0)),
        out_shape=jax.ShapeDtypeStruct((_B, _OUT), jnp.float32),
    )(x, local, W1, b1.reshape(1, _HID), W2, b2.reshape(1, _OUT))


def kernel(x, edge_index, agent_nodes, W1, b1, W2, b2):
    a = agent_nodes.astype(jnp.int32)
    idx = a[:, 0] * _N + a[:, 1]
    table = x[:_N].reshape(_N * _N, _D)
    local = _make_sc_gather()(table, idx)
    return _tc_call(x, local, W1, b1, W2, b2)


# speedup vs baseline: 1.0058x; 1.0058x over previous
import functools

import jax
import jax.numpy as jnp
from jax import lax
from jax.experimental import pallas as pl
from jax.experimental.pallas import tpu as pltpu
from jax.experimental.pallas import tpu_sc as plsc

_B, _N, _D = 4096, 50, 256
_HID = 2 * _D
_OUT = 256

_NC, _NS = 2, 16
_NW = _NC * _NS
_B_PER_W = _B // _NW


@functools.cache
def _make_sc_gather():
    mesh = plsc.VectorSubcoreMesh(core_axis_name="c", subcore_axis_name="s")

    @functools.partial(
        pl.kernel,
        mesh=mesh,
        out_type=jax.ShapeDtypeStruct((_B, _D), jnp.float32),
        scratch_types=[
            pltpu.VMEM((_B_PER_W,), jnp.int32),
            pltpu.VMEM((_B_PER_W, _D), jnp.float32),
            pltpu.SemaphoreType.DMA,
        ],
    )
    def gather_k(table_hbm, idx_hbm, out_hbm, idx_v, rows_v, sem):
        wid = lax.axis_index("s") * _NC + lax.axis_index("c")
        base = wid * _B_PER_W
        pltpu.sync_copy(idx_hbm.at[pl.ds(base, _B_PER_W)], idx_v)
        pltpu.async_copy(table_hbm.at[idx_v], rows_v, sem).wait()
        pltpu.sync_copy(rows_v, out_hbm.at[pl.ds(base, _B_PER_W)])

    return gather_k


_BB = 64
_NX = 4


def _tc_body(*refs):
    x_refs = refs[:_NX]
    local_ref, w1_ref, b1_ref, w2_ref, b2_ref, o_ref = refs[_NX:]
    means = [jnp.mean(r[...], axis=1) for r in x_refs]
    mean = jnp.concatenate(means, axis=0)
    h = jnp.concatenate([local_ref[...], mean], axis=1)
    h = jnp.dot(h, w1_ref[...], preferred_element_type=jnp.float32)
    h = jnp.maximum(h + b1_ref[...], 0.0)
    h = jnp.dot(h, w2_ref[...], preferred_element_type=jnp.float32)
    o_ref[...] = jnp.maximum(h + b2_ref[...], 0.0)


def _tc_call(x, local, W1, b1, W2, b2):
    grid = (_B // (_BB * _NX),)

    def _x_spec(k):
        return pl.BlockSpec((_BB, _N, _D), lambda i, k=k: (i * _NX + k, 0, 0))

    return pl.pallas_call(
        _tc_body,
        grid=grid,
        in_specs=[_x_spec(k) for k in range(_NX)]
        + [
            pl.BlockSpec((_NX * _BB, _D), lambda i: (i, 0)),
            pl.BlockSpec((_HID, _HID), lambda i: (0, 0)),
            pl.BlockSpec((1, _HID), lambda i: (0, 0)),
            pl.BlockSpec((_HID, _OUT), lambda i: (0, 0)),
            pl.BlockSpec((1, _OUT), lambda i: (0, 0)),
        ],
        out_specs=pl.BlockSpec((_NX * _BB, _OUT), lambda i: (i, 0)),
        out_shape=jax.ShapeDtypeStruct((_B, _OUT), jnp.float32),
    )(*([x] * _NX), local, W1, b1.reshape(1, _HID), W2, b2.reshape(1, _OUT))


def kernel(x, edge_index, agent_nodes, W1, b1, W2, b2):
    a = agent_nodes.astype(jnp.int32)
    idx = a[:, 0] * _N + a[:, 1]
    table = x[:_N].reshape(_N * _N, _D)
    local = _make_sc_gather()(table, idx)
    return _tc_call(x, local, W1, b1, W2, b2)


# speedup vs baseline: 2.8113x; 2.7950x over previous
import functools

import jax
import jax.numpy as jnp
from jax import lax
from jax.experimental import pallas as pl
from jax.experimental.pallas import tpu as pltpu
from jax.experimental.pallas import tpu_sc as plsc

_B, _N, _D = 4096, 50, 256
_HID = 2 * _D
_OUT = 256

_NC, _NS = 2, 16
_NW = _NC * _NS
_B_PER_W = _B // _NW


@functools.cache
def _make_sc_gather():
    mesh = plsc.VectorSubcoreMesh(core_axis_name="c", subcore_axis_name="s")

    @functools.partial(
        pl.kernel,
        mesh=mesh,
        out_type=jax.ShapeDtypeStruct((_B, _D), jnp.float32),
        scratch_types=[
            pltpu.VMEM((_B_PER_W,), jnp.int32),
            pltpu.VMEM((_B_PER_W, _D), jnp.float32),
            pltpu.SemaphoreType.DMA,
        ],
    )
    def gather_k(table_hbm, idx_hbm, out_hbm, idx_v, rows_v, sem):
        wid = lax.axis_index("s") * _NC + lax.axis_index("c")
        base = wid * _B_PER_W
        pltpu.sync_copy(idx_hbm.at[pl.ds(base, _B_PER_W)], idx_v)
        pltpu.async_copy(table_hbm.at[idx_v], rows_v, sem).wait()
        pltpu.sync_copy(rows_v, out_hbm.at[pl.ds(base, _B_PER_W)])

    return gather_k


_BB = 256


def _tc_body(xt_ref, local_ref, w1_ref, b1_ref, w2_ref, b2_ref, o_ref):
    mean = jnp.mean(xt_ref[...], axis=0)
    h = jnp.concatenate([local_ref[...], mean], axis=1)
    h = jnp.dot(h, w1_ref[...], preferred_element_type=jnp.float32)
    h = jnp.maximum(h + b1_ref[...], 0.0)
    h = jnp.dot(h, w2_ref[...], preferred_element_type=jnp.float32)
    o_ref[...] = jnp.maximum(h + b2_ref[...], 0.0)


def _tc_call(xt, local, W1, b1, W2, b2):
    grid = (_B // _BB,)
    return pl.pallas_call(
        _tc_body,
        grid=grid,
        in_specs=[
            pl.BlockSpec((_N, _BB, _D), lambda i: (0, i, 0)),
            pl.BlockSpec((_BB, _D), lambda i: (i, 0)),
            pl.BlockSpec((_HID, _HID), lambda i: (0, 0)),
            pl.BlockSpec((1, _HID), lambda i: (0, 0)),
            pl.BlockSpec((_HID, _OUT), lambda i: (0, 0)),
            pl.BlockSpec((1, _OUT), lambda i: (0, 0)),
        ],
        out_specs=pl.BlockSpec((_BB, _OUT), lambda i: (i, 0)),
        out_shape=jax.ShapeDtypeStruct((_B, _OUT), jnp.float32),
    )(xt, local, W1, b1.reshape(1, _HID), W2, b2.reshape(1, _OUT))


def kernel(x, edge_index, agent_nodes, W1, b1, W2, b2):
    a = agent_nodes.astype(jnp.int32)
    xt = jnp.transpose(x, (1, 0, 2))
    table = xt.reshape(_N * _B, _D)
    idx = a[:, 1] * _B + a[:, 0]
    local = _make_sc_gather()(table, idx)
    return _tc_call(xt, local, W1, b1, W2, b2)
